# baseline (speedup 1.0000x reference)
import jax
import jax.numpy as jnp
from jax.experimental import pallas as pl
from jax.experimental.pallas import tpu as pltpu

TILE = 4096


def _mlp_kernel(x_ref, wp_ref, bp_ref, w1_ref, b1_ref, w2_ref, b2_ref,
                o_ref, we_ref, be_ref):
    @pl.when(pl.program_id(0) == 0)
    def _fold():
        w1 = w1_ref[...]
        we_ref[...] = jnp.dot(wp_ref[...], w1, preferred_element_type=jnp.float32)
        be_ref[...] = (
            jnp.dot(bp_ref[...], w1, preferred_element_type=jnp.float32)
            + b1_ref[...]
        )

    SUB = 1024
    for s in range(TILE // SUB):
        x = x_ref[pl.ds(s * SUB, SUB), :]
        h = jnp.dot(x, we_ref[...],
                    preferred_element_type=jnp.float32) + be_ref[...]
        h = jnp.maximum(h, 0.0)
        o_ref[pl.ds(s * SUB, SUB), :] = jnp.dot(
            h, w2_ref[...], preferred_element_type=jnp.float32) + b2_ref[...]


def kernel(trial_feats, Wp, bp, W1, b1, W2, b2):
    B, F = trial_feats.shape
    H = Wp.shape[1]
    O = W2.shape[1]
    grid = (B // TILE,)
    return pl.pallas_call(
        _mlp_kernel,
        grid=grid,
        in_specs=[
            pl.BlockSpec((TILE, F), lambda i: (i, 0)),
            pl.BlockSpec((F, H), lambda i: (0, 0)),
            pl.BlockSpec((1, H), lambda i: (0, 0)),
            pl.BlockSpec((H, H), lambda i: (0, 0)),
            pl.BlockSpec((1, H), lambda i: (0, 0)),
            pl.BlockSpec((H, O), lambda i: (0, 0)),
            pl.BlockSpec((1, O), lambda i: (0, 0)),
        ],
        out_specs=pl.BlockSpec((TILE, O), lambda i: (i, 0)),
        out_shape=jax.ShapeDtypeStruct((B, O), jnp.float32),
        scratch_shapes=[
            pltpu.VMEM((F, H), jnp.float32),
            pltpu.VMEM((1, H), jnp.float32),
        ],
        compiler_params=pltpu.CompilerParams(
            dimension_semantics=("arbitrary",),
        ),
    )(trial_feats, Wp, bp.reshape(1, H), W1, b1.reshape(1, H),
      W2, b2.reshape(1, O))


# speedup vs baseline: 1.0453x; 1.0453x over previous
import jax
import jax.numpy as jnp
from jax.experimental import pallas as pl
from jax.experimental.pallas import tpu as pltpu

TILE = 4096


def _mlp_kernel(x_ref, wp_ref, bp_ref, w1_ref, b1_ref, w2_ref, b2_ref,
                o_ref, we_ref, be_ref):
    @pl.when(pl.program_id(0) == 0)
    def _fold():
        w1 = w1_ref[...]
        we_ref[...] = jnp.dot(wp_ref[...], w1, preferred_element_type=jnp.float32)
        be_ref[...] = (
            jnp.dot(bp_ref[...], w1, preferred_element_type=jnp.float32)
            + b1_ref[...]
        )

    h = jnp.dot(x_ref[...], we_ref[...],
                preferred_element_type=jnp.float32) + be_ref[...]
    h = jnp.maximum(h, 0.0)
    i = pl.program_id(0)
    o_ref[pl.ds(i * TILE, TILE), :] = jnp.dot(
        h, w2_ref[...], preferred_element_type=jnp.float32) + b2_ref[...]


def kernel(trial_feats, Wp, bp, W1, b1, W2, b2):
    B, F = trial_feats.shape
    H = Wp.shape[1]
    O = W2.shape[1]
    grid = (B // TILE,)
    return pl.pallas_call(
        _mlp_kernel,
        grid=grid,
        in_specs=[
            pl.BlockSpec((TILE, F), lambda i: (i, 0)),
            pl.BlockSpec((F, H), lambda i: (0, 0)),
            pl.BlockSpec((1, H), lambda i: (0, 0)),
            pl.BlockSpec((H, H), lambda i: (0, 0)),
            pl.BlockSpec((1, H), lambda i: (0, 0)),
            pl.BlockSpec((H, O), lambda i: (0, 0)),
            pl.BlockSpec((1, O), lambda i: (0, 0)),
        ],
        out_specs=pl.BlockSpec(memory_space=pltpu.MemorySpace.VMEM),
        out_shape=jax.ShapeDtypeStruct((B, O), jnp.float32),
        scratch_shapes=[
            pltpu.VMEM((F, H), jnp.float32),
            pltpu.VMEM((1, H), jnp.float32),
        ],
        compiler_params=pltpu.CompilerParams(
            dimension_semantics=("arbitrary",),
        ),
    )(trial_feats, Wp, bp.reshape(1, H), W1, b1.reshape(1, H),
      W2, b2.reshape(1, O))


# speedup vs baseline: 1.0729x; 1.0264x over previous
import jax
import jax.numpy as jnp
from jax.experimental import pallas as pl
from jax.experimental.pallas import tpu as pltpu

TILE = 4096


def _mlp_kernel(x_ref, wp_ref, bp_ref, w1_ref, b1_ref, w2_ref, b2_ref,
                o_ref, we_ref, be_ref):
    @pl.when(pl.program_id(0) == 0)
    def _fold():
        w1 = w1_ref[...]
        we_ref[...] = jnp.dot(wp_ref[...], w1, preferred_element_type=jnp.float32)
        be_ref[...] = (
            jnp.dot(bp_ref[...], w1, preferred_element_type=jnp.float32)
            + b1_ref[...]
        )

    h = jnp.dot(x_ref[...], we_ref[...],
                preferred_element_type=jnp.float32) + be_ref[...]
    h = jnp.maximum(h, 0.0)
    o_ref[...] = jnp.dot(h, w2_ref[...],
                         preferred_element_type=jnp.float32) + b2_ref[...]


def kernel(trial_feats, Wp, bp, W1, b1, W2, b2):
    B, F = trial_feats.shape
    H = Wp.shape[1]
    O = W2.shape[1]
    grid = (B // TILE,)
    return pl.pallas_call(
        _mlp_kernel,
        grid=grid,
        in_specs=[
            pl.BlockSpec((TILE, F), lambda i: (i, 0)),
            pl.BlockSpec((F, H), lambda i: (0, 0)),
            pl.BlockSpec((1, H), lambda i: (0, 0)),
            pl.BlockSpec((H, H), lambda i: (0, 0)),
            pl.BlockSpec((1, H), lambda i: (0, 0)),
            pl.BlockSpec((H, O), lambda i: (0, 0)),
            pl.BlockSpec((1, O), lambda i: (0, 0)),
        ],
        out_specs=pl.BlockSpec((TILE, O), lambda i: (i, 0)),
        out_shape=jax.ShapeDtypeStruct((B, O), jnp.float32),
        scratch_shapes=[
            pltpu.VMEM((F, H), jnp.float32),
            pltpu.VMEM((1, H), jnp.float32),
        ],
        compiler_params=pltpu.CompilerParams(
            dimension_semantics=("arbitrary",),
        ),
    )(trial_feats, Wp, bp.reshape(1, H), W1, b1.reshape(1, H),
      W2, b2.reshape(1, O))
